# baseline (speedup 1.0000x reference)
import jax
import jax.numpy as jnp
from jax.experimental import pallas as pl

N, C, T, V, M = 16, 4, 128, 25, 10
T4 = T // 4
FEAT_DIM, NUM_CLASS = 256, 60
K_NN = 4
LAMBDA_FUSE = 0.1
JTV = T4 * V * M


def _fused_kernel(xs_ref, hips_ref, Wf_ref, bf_ref, WbaseT_ref, bbase_ref,
                  projWT_ref, projb_ref, tagWT_ref, tagb_ref, tsc_ref,
                  out_ref):
    f32 = jnp.float32
    xs = xs_ref[0]
    feat = jnp.dot(Wf_ref[...], xs, preferred_element_type=f32)
    feat = jnp.maximum(feat + bf_ref[...], 0.0)

    col_m = jax.lax.broadcasted_iota(jnp.int32, (JTV, M), 0) % M
    sel_m = jax.lax.broadcasted_iota(jnp.int32, (JTV, M), 1)
    sel = (col_m == sel_m).astype(f32)
    S = jnp.dot(feat, sel, preferred_element_type=f32)

    pooled = jnp.sum(S, axis=1, keepdims=True).T / float(JTV)
    pf = S.T / float(T4 * V)

    logits_base = jnp.dot(pooled, WbaseT_ref[...],
                          preferred_element_type=f32) + bbase_ref[...]

    hips = hips_ref[0]
    hsum = jnp.sum(hips, axis=1) / float(T)
    pos = 0.5 * (hsum[:, :M] + hsum[:, M:])

    d2 = jnp.zeros((M, M), dtype=f32)
    for c in range(3):
        row = pos[c:c + 1, :]
        colv = row.T
        diff = colv - row
        d2 = d2 + diff * diff
    d = jnp.sqrt(d2)

    col_idx = jax.lax.broadcasted_iota(jnp.int32, (M, M), 1)
    rank = jnp.zeros((M, M), dtype=jnp.int32)
    for jp in range(M):
        dj = d[:, jp:jp + 1]
        lt = dj < d
        eq = (dj == d) & (jp < col_idx)
        rank = rank + (lt | eq).astype(jnp.int32)
    row_idx = jax.lax.broadcasted_iota(jnp.int32, (M, M), 0)
    A = (rank < K_NN).astype(f32) + (row_idx == col_idx).astype(f32)
    A = A / (jnp.sum(A, axis=1, keepdims=True) + 1e-6)

    h = jnp.dot(pf, projWT_ref[...], preferred_element_type=f32) + projb_ref[...]
    h = h + LAMBDA_FUSE * jnp.dot(A, h, preferred_element_type=f32)
    h = jnp.maximum(h, 0.0)
    hbar = jnp.sum(h, axis=0, keepdims=True) / float(M)
    logits_tag = jnp.dot(hbar, tagWT_ref[...],
                         preferred_element_type=f32) + tagb_ref[...]

    gate = jax.nn.sigmoid(tsc_ref[0, 0])
    out_ref[0] = logits_base + gate * logits_tag


def kernel(x, Wf, bf, Wbase, bbase, proj_W, proj_b, tag_W, tag_b, tag_scale):
    f32 = jnp.float32
    xs = x[:, :, ::4, :, :].reshape(N, C, JTV)
    hips = x[:, 0:3, :, 11:13, :].reshape(N, 3, T, 2 * M)
    bf_c = bf.reshape(FEAT_DIM, 1)
    WbaseT = Wbase.T
    bbase_r = bbase.reshape(1, NUM_CLASS)
    projWT = proj_W.T
    projb_r = proj_b.reshape(1, FEAT_DIM)
    tagWT = tag_W.T
    tagb_r = tag_b.reshape(1, NUM_CLASS)
    tsc = tag_scale.reshape(1, 1).astype(f32)

    out3 = pl.pallas_call(
        _fused_kernel,
        grid=(N,),
        in_specs=[
            pl.BlockSpec((1, C, JTV), lambda n: (n, 0, 0)),
            pl.BlockSpec((1, 3, T, 2 * M), lambda n: (n, 0, 0, 0)),
            pl.BlockSpec((FEAT_DIM, C), lambda n: (0, 0)),
            pl.BlockSpec((FEAT_DIM, 1), lambda n: (0, 0)),
            pl.BlockSpec((FEAT_DIM, NUM_CLASS), lambda n: (0, 0)),
            pl.BlockSpec((1, NUM_CLASS), lambda n: (0, 0)),
            pl.BlockSpec((FEAT_DIM, FEAT_DIM), lambda n: (0, 0)),
            pl.BlockSpec((1, FEAT_DIM), lambda n: (0, 0)),
            pl.BlockSpec((FEAT_DIM, NUM_CLASS), lambda n: (0, 0)),
            pl.BlockSpec((1, NUM_CLASS), lambda n: (0, 0)),
            pl.BlockSpec((1, 1), lambda n: (0, 0)),
        ],
        out_specs=pl.BlockSpec((1, 1, NUM_CLASS), lambda n: (n, 0, 0)),
        out_shape=jax.ShapeDtypeStruct((N, 1, NUM_CLASS), f32),
    )(xs, hips, Wf, bf_c, WbaseT, bbase_r, projWT, projb_r, tagWT, tagb_r, tsc)
    return out3.reshape(N, NUM_CLASS)


# speedup vs baseline: 1.2358x; 1.2358x over previous
import functools

import jax
import jax.numpy as jnp
from jax.experimental import pallas as pl

N, C, T, V, M = 16, 4, 128, 25, 10
T4 = T // 4
FEAT_DIM, NUM_CLASS = 256, 60
K_NN = 4
LAMBDA_FUSE = 0.1
JVM = V * M
HIP_L, HIP_R = 11, 12


def _fused_kernel(x_ref, hips_ref, Wf_ref, bf_ref, WbaseT_ref,
                  bbase_ref, projWT_ref, projb_ref, tagWT_ref, tagb_ref,
                  tsc_ref, out_ref):
    f32 = jnp.float32

    Wf = Wf_ref[...]
    bf = bf_ref[...]
    col_m = jax.lax.broadcasted_iota(jnp.int32, (JVM, M), 0) % M
    sel_m = jax.lax.broadcasted_iota(jnp.int32, (JVM, M), 1)
    sel = (col_m == sel_m).astype(f32)

    def body(i, S):
        xt = x_ref[0, :, i, 0].reshape(C, JVM)
        feat = jnp.dot(Wf, xt, preferred_element_type=f32)
        feat = jnp.maximum(feat + bf, 0.0)
        return S + jnp.dot(feat, sel, preferred_element_type=f32)

    S = jax.lax.fori_loop(0, T4, body, jnp.zeros((FEAT_DIM, M), f32))

    pooled = jnp.sum(S, axis=1, keepdims=True).T / float(T4 * JVM)
    pf = S.T / float(T4 * V)
    logits_base = jnp.dot(pooled, WbaseT_ref[...],
                          preferred_element_type=f32) + bbase_ref[...]

    hips = hips_ref[0]
    hsum = jnp.sum(hips, axis=1) / float(T)
    pos = 0.5 * (hsum[:, :M] + hsum[:, M:])

    d2 = jnp.zeros((M, M), dtype=f32)
    for c in range(3):
        row = pos[c:c + 1, :]
        diff = row.T - row
        d2 = d2 + diff * diff
    d = jnp.sqrt(d2)

    col_idx = jax.lax.broadcasted_iota(jnp.int32, (M, M), 1)
    rank = jnp.zeros((M, M), dtype=jnp.int32)
    for jp in range(M):
        dj = d[:, jp:jp + 1]
        hit = (dj < d) | ((dj == d) & (jp < col_idx))
        rank = rank + hit.astype(jnp.int32)
    row_idx = jax.lax.broadcasted_iota(jnp.int32, (M, M), 0)
    A = (rank < K_NN).astype(f32) + (row_idx == col_idx).astype(f32)
    A = A / (jnp.sum(A, axis=1, keepdims=True) + 1e-6)

    h = jnp.dot(pf, projWT_ref[...], preferred_element_type=f32) + projb_ref[...]
    h = h + LAMBDA_FUSE * jnp.dot(A, h, preferred_element_type=f32)
    h = jnp.maximum(h, 0.0)
    hbar = jnp.sum(h, axis=0, keepdims=True) / float(M)
    logits_tag = jnp.dot(hbar, tagWT_ref[...],
                         preferred_element_type=f32) + tagb_ref[...]

    gate = jax.nn.sigmoid(tsc_ref[0, 0])
    out_ref[0] = logits_base + gate * logits_tag


def kernel(x, Wf, bf, Wbase, bbase, proj_W, proj_b, tag_W, tag_b, tag_scale):
    f32 = jnp.float32
    x6 = x.reshape(N, C, T4, 4, V, M)
    hips = x[:, 0:3, :, HIP_L:HIP_R + 1, :].reshape(N, 3, T, 2 * M)
    bf_c = bf.reshape(FEAT_DIM, 1)
    WbaseT = Wbase.T
    bbase_r = bbase.reshape(1, NUM_CLASS)
    projWT = proj_W.T
    projb_r = proj_b.reshape(1, FEAT_DIM)
    tagWT = tag_W.T
    tagb_r = tag_b.reshape(1, NUM_CLASS)
    tsc = tag_scale.reshape(1, 1).astype(f32)

    out3 = pl.pallas_call(
        _fused_kernel,
        grid=(N,),
        in_specs=[
            pl.BlockSpec((1, C, T4, 1, V, M), lambda n: (n, 0, 0, 0, 0, 0)),
            pl.BlockSpec((1, 3, T, 2 * M), lambda n: (n, 0, 0, 0)),
            pl.BlockSpec((FEAT_DIM, C), lambda n: (0, 0)),
            pl.BlockSpec((FEAT_DIM, 1), lambda n: (0, 0)),
            pl.BlockSpec((FEAT_DIM, NUM_CLASS), lambda n: (0, 0)),
            pl.BlockSpec((1, NUM_CLASS), lambda n: (0, 0)),
            pl.BlockSpec((FEAT_DIM, FEAT_DIM), lambda n: (0, 0)),
            pl.BlockSpec((1, FEAT_DIM), lambda n: (0, 0)),
            pl.BlockSpec((FEAT_DIM, NUM_CLASS), lambda n: (0, 0)),
            pl.BlockSpec((1, NUM_CLASS), lambda n: (0, 0)),
            pl.BlockSpec((1, 1), lambda n: (0, 0)),
        ],
        out_specs=pl.BlockSpec((1, 1, NUM_CLASS), lambda n: (n, 0, 0)),
        out_shape=jax.ShapeDtypeStruct((N, 1, NUM_CLASS), f32),
    )(x6, hips, Wf, bf_c, WbaseT, bbase_r, projWT, projb_r, tagWT, tagb_r, tsc)
    return out3.reshape(N, NUM_CLASS)


# speedup vs baseline: 3.4280x; 2.7739x over previous
import functools

import jax
import jax.numpy as jnp
from jax.experimental import pallas as pl

N, C, T, V, M = 16, 4, 128, 25, 10
T4 = T // 4
FEAT_DIM, NUM_CLASS = 256, 60
K_NN = 4
LAMBDA_FUSE = 0.1
JVM = V * M
HIP_L, HIP_R = 11, 12


def _fused_kernel(x_ref, hips_ref, Wf_ref, bf_ref, WbaseT_ref,
                  bbase_ref, projWT_ref, projb_ref, tagWT_ref, tagb_ref,
                  tsc_ref, out_ref):
    f32 = jnp.float32

    JALL = T4 * JVM
    xmat = x_ref[0, :, :, 0].reshape(C, JALL)
    feat = jnp.dot(Wf_ref[...], xmat, preferred_element_type=f32)
    feat = jnp.maximum(feat + bf_ref[...], 0.0)

    col_m = jax.lax.broadcasted_iota(jnp.int32, (JALL, M), 0) % M
    sel_m = jax.lax.broadcasted_iota(jnp.int32, (JALL, M), 1)
    sel = (col_m == sel_m).astype(f32)
    S = jnp.dot(feat, sel, preferred_element_type=f32)

    pooled = jnp.sum(S, axis=1, keepdims=True).T / float(T4 * JVM)
    pf = S.T / float(T4 * V)
    logits_base = jnp.dot(pooled, WbaseT_ref[...],
                          preferred_element_type=f32) + bbase_ref[...]

    hips = hips_ref[0]
    hsum = jnp.sum(hips, axis=1) / float(T)
    pos = 0.5 * (hsum[:, :M] + hsum[:, M:])

    d2 = jnp.zeros((M, M), dtype=f32)
    for c in range(3):
        row = pos[c:c + 1, :]
        diff = row.T - row
        d2 = d2 + diff * diff
    d = jnp.sqrt(d2)

    col_idx = jax.lax.broadcasted_iota(jnp.int32, (M, M), 1)
    rank = jnp.zeros((M, M), dtype=jnp.int32)
    for jp in range(M):
        dj = d[:, jp:jp + 1]
        hit = (dj < d) | ((dj == d) & (jp < col_idx))
        rank = rank + hit.astype(jnp.int32)
    row_idx = jax.lax.broadcasted_iota(jnp.int32, (M, M), 0)
    A = (rank < K_NN).astype(f32) + (row_idx == col_idx).astype(f32)
    A = A / (jnp.sum(A, axis=1, keepdims=True) + 1e-6)

    h = jnp.dot(pf, projWT_ref[...], preferred_element_type=f32) + projb_ref[...]
    h = h + LAMBDA_FUSE * jnp.dot(A, h, preferred_element_type=f32)
    h = jnp.maximum(h, 0.0)
    hbar = jnp.sum(h, axis=0, keepdims=True) / float(M)
    logits_tag = jnp.dot(hbar, tagWT_ref[...],
                         preferred_element_type=f32) + tagb_ref[...]

    gate = jax.nn.sigmoid(tsc_ref[0, 0])
    out_ref[0] = logits_base + gate * logits_tag


def kernel(x, Wf, bf, Wbase, bbase, proj_W, proj_b, tag_W, tag_b, tag_scale):
    f32 = jnp.float32
    x6 = x.reshape(N, C, T4, 4, V, M)
    hips = x[:, 0:3, :, HIP_L:HIP_R + 1, :].reshape(N, 3, T, 2 * M)
    bf_c = bf.reshape(FEAT_DIM, 1)
    WbaseT = Wbase.T
    bbase_r = bbase.reshape(1, NUM_CLASS)
    projWT = proj_W.T
    projb_r = proj_b.reshape(1, FEAT_DIM)
    tagWT = tag_W.T
    tagb_r = tag_b.reshape(1, NUM_CLASS)
    tsc = tag_scale.reshape(1, 1).astype(f32)

    out3 = pl.pallas_call(
        _fused_kernel,
        grid=(N,),
        in_specs=[
            pl.BlockSpec((1, C, T4, 1, V, M), lambda n: (n, 0, 0, 0, 0, 0)),
            pl.BlockSpec((1, 3, T, 2 * M), lambda n: (n, 0, 0, 0)),
            pl.BlockSpec((FEAT_DIM, C), lambda n: (0, 0)),
            pl.BlockSpec((FEAT_DIM, 1), lambda n: (0, 0)),
            pl.BlockSpec((FEAT_DIM, NUM_CLASS), lambda n: (0, 0)),
            pl.BlockSpec((1, NUM_CLASS), lambda n: (0, 0)),
            pl.BlockSpec((FEAT_DIM, FEAT_DIM), lambda n: (0, 0)),
            pl.BlockSpec((1, FEAT_DIM), lambda n: (0, 0)),
            pl.BlockSpec((FEAT_DIM, NUM_CLASS), lambda n: (0, 0)),
            pl.BlockSpec((1, NUM_CLASS), lambda n: (0, 0)),
            pl.BlockSpec((1, 1), lambda n: (0, 0)),
        ],
        out_specs=pl.BlockSpec((1, 1, NUM_CLASS), lambda n: (n, 0, 0)),
        out_shape=jax.ShapeDtypeStruct((N, 1, NUM_CLASS), f32),
    )(x6, hips, Wf, bf_c, WbaseT, bbase_r, projWT, projb_r, tagWT, tagb_r, tsc)
    return out3.reshape(N, NUM_CLASS)


# speedup vs baseline: 3.8103x; 1.1115x over previous
import jax
import jax.numpy as jnp
from jax.experimental import pallas as pl

N, C, T, V, M = 16, 4, 128, 25, 10
T4 = T // 4
FEAT_DIM, NUM_CLASS = 256, 60
K_NN = 4
LAMBDA_FUSE = 0.1
JVM = V * M
JALL = T4 * JVM
HIP_L, HIP_R = 11, 12
VSUB = 8


def _fused_kernel(x_ref, hv_ref, Wf_ref, bf_ref, WbaseT_ref,
                  bbase_ref, projWT_ref, projb_ref, tagWT_ref, tagb_ref,
                  tsc_ref, out_ref):
    f32 = jnp.float32
    bf16 = jnp.bfloat16

    xmat = x_ref[0, :, :, 0].reshape(C, JALL).astype(bf16)
    feat = jnp.dot(Wf_ref[...].astype(bf16), xmat,
                   preferred_element_type=f32)
    feat = jnp.maximum(feat + bf_ref[...], 0.0)

    col_m = jax.lax.broadcasted_iota(jnp.int32, (M, JALL), 1) % M
    sel_m = jax.lax.broadcasted_iota(jnp.int32, (M, JALL), 0)
    selT = (col_m == sel_m).astype(f32)
    ST = jax.lax.dot_general(selT, feat, (((1,), (1,)), ((), ())),
                             preferred_element_type=f32)
    pf = ST / float(T4 * V)
    pooled = jnp.sum(pf, axis=0, keepdims=True) * (float(T4 * V) / float(JALL))
    logits_base = jnp.dot(pooled, WbaseT_ref[...],
                          preferred_element_type=f32) + bbase_ref[...]

    hv = hv_ref[0]
    hsel = hv[:, :, :, HIP_L - VSUB:HIP_R - VSUB + 1, :]
    hsum = jnp.sum(hsel, axis=(1, 2)) / float(T)
    pos = 0.5 * (hsum[:, 0, :] + hsum[:, 1, :])

    d2 = jnp.zeros((M, M), dtype=f32)
    for c in range(3):
        row = pos[c:c + 1, :]
        diff = row.T - row
        d2 = d2 + diff * diff
    d = jnp.sqrt(d2)

    col_idx = jax.lax.broadcasted_iota(jnp.int32, (M, M), 1)
    rank = jnp.zeros((M, M), dtype=jnp.int32)
    for jp in range(M):
        dj = d[:, jp:jp + 1]
        hit = (dj < d) | ((dj == d) & (jp < col_idx))
        rank = rank + hit.astype(jnp.int32)
    row_idx = jax.lax.broadcasted_iota(jnp.int32, (M, M), 0)
    A = (rank < K_NN).astype(f32) + (row_idx == col_idx).astype(f32)
    A = A / (jnp.sum(A, axis=1, keepdims=True) + 1e-6)

    h = jnp.dot(pf, projWT_ref[...], preferred_element_type=f32) + projb_ref[...]
    h = h + LAMBDA_FUSE * jnp.dot(A, h, preferred_element_type=f32)
    h = jnp.maximum(h, 0.0)
    hbar = jnp.sum(h, axis=0, keepdims=True) / float(M)
    logits_tag = jnp.dot(hbar, tagWT_ref[...],
                         preferred_element_type=f32) + tagb_ref[...]

    gate = jax.nn.sigmoid(tsc_ref[0, 0])
    out_ref[0] = logits_base + gate * logits_tag


def kernel(x, Wf, bf, Wbase, bbase, proj_W, proj_b, tag_W, tag_b, tag_scale):
    f32 = jnp.float32
    x6 = x.reshape(N, C, T4, 4, V, M)
    bf_c = bf.reshape(FEAT_DIM, 1)
    WbaseT = Wbase.T
    bbase_r = bbase.reshape(1, NUM_CLASS)
    projWT = proj_W.T
    projb_r = proj_b.reshape(1, FEAT_DIM)
    tagWT = tag_W.T
    tagb_r = tag_b.reshape(1, NUM_CLASS)
    tsc = tag_scale.reshape(1, 1).astype(f32)

    out3 = pl.pallas_call(
        _fused_kernel,
        grid=(N,),
        in_specs=[
            pl.BlockSpec((1, C, T4, 1, V, M), lambda n: (n, 0, 0, 0, 0, 0)),
            pl.BlockSpec((1, 3, T4, 4, VSUB, M),
                         lambda n: (n, 0, 0, 0, 1, 0)),
            pl.BlockSpec((FEAT_DIM, C), lambda n: (0, 0)),
            pl.BlockSpec((FEAT_DIM, 1), lambda n: (0, 0)),
            pl.BlockSpec((FEAT_DIM, NUM_CLASS), lambda n: (0, 0)),
            pl.BlockSpec((1, NUM_CLASS), lambda n: (0, 0)),
            pl.BlockSpec((FEAT_DIM, FEAT_DIM), lambda n: (0, 0)),
            pl.BlockSpec((1, FEAT_DIM), lambda n: (0, 0)),
            pl.BlockSpec((FEAT_DIM, NUM_CLASS), lambda n: (0, 0)),
            pl.BlockSpec((1, NUM_CLASS), lambda n: (0, 0)),
            pl.BlockSpec((1, 1), lambda n: (0, 0)),
        ],
        out_specs=pl.BlockSpec((1, 1, NUM_CLASS), lambda n: (n, 0, 0)),
        out_shape=jax.ShapeDtypeStruct((N, 1, NUM_CLASS), f32),
    )(x6, x6, Wf, bf_c, WbaseT, bbase_r, projWT, projb_r, tagWT, tagb_r, tsc)
    return out3.reshape(N, NUM_CLASS)


# speedup vs baseline: 3.8785x; 1.0179x over previous
import jax
import jax.numpy as jnp
from jax.experimental import pallas as pl

N, C, T, V, M = 16, 4, 128, 25, 10
T4 = T // 4
FEAT_DIM, NUM_CLASS = 256, 60
K_NN = 4
LAMBDA_FUSE = 0.1
JVM = V * M
JALL = T4 * JVM
G = V * M // 2
HIP_L, HIP_R = 11, 12
GB = 5


def _fused_kernel(x_ref, ha_ref, hb_ref, Wf_ref, bf_ref, WbaseT_ref,
                  bbase_ref, projWT_ref, projb_ref, tagWT_ref, tagb_ref,
                  tsc_ref, out_ref):
    f32 = jnp.float32
    bf16 = jnp.bfloat16

    xmat = x_ref[0, :, :, 0].reshape(C, JALL).astype(bf16)
    feat = jnp.dot(Wf_ref[...].astype(bf16), xmat,
                   preferred_element_type=f32)
    feat = jnp.maximum(feat + bf_ref[...], 0.0)

    col_m = jax.lax.broadcasted_iota(jnp.int32, (M, JALL), 1) % M
    sel_m = jax.lax.broadcasted_iota(jnp.int32, (M, JALL), 0)
    selT = (col_m == sel_m).astype(f32)
    ST = jax.lax.dot_general(selT, feat, (((1,), (1,)), ((), ())),
                             preferred_element_type=f32)
    pf = ST / float(T4 * V)
    pooled = jnp.sum(pf, axis=0, keepdims=True) / float(M)
    logits_base = jnp.dot(pooled, WbaseT_ref[...],
                          preferred_element_type=f32) + bbase_ref[...]

    def joint_pos(ref):
        s = jnp.sum(ref[0], axis=2) / float(T)
        return s.reshape(GB, 2, 4).transpose(2, 0, 1).reshape(4, M)

    pos = 0.5 * (joint_pos(ha_ref) + joint_pos(hb_ref))
    pos = pos[0:3]

    d2 = jnp.zeros((M, M), dtype=f32)
    for c in range(3):
        row = pos[c:c + 1, :]
        diff = row.T - row
        d2 = d2 + diff * diff
    d = jnp.sqrt(d2)

    col_idx = jax.lax.broadcasted_iota(jnp.int32, (M, M), 1)
    rank = jnp.zeros((M, M), dtype=jnp.int32)
    for jp in range(M):
        dj = d[:, jp:jp + 1]
        hit = (dj < d) | ((dj == d) & (jp < col_idx))
        rank = rank + hit.astype(jnp.int32)
    row_idx = jax.lax.broadcasted_iota(jnp.int32, (M, M), 0)
    A = (rank < K_NN).astype(f32) + (row_idx == col_idx).astype(f32)
    A = A / (jnp.sum(A, axis=1, keepdims=True) + 1e-6)

    h = jnp.dot(pf, projWT_ref[...], preferred_element_type=f32) + projb_ref[...]
    h = h + LAMBDA_FUSE * jnp.dot(A, h, preferred_element_type=f32)
    h = jnp.maximum(h, 0.0)
    hbar = jnp.sum(h, axis=0, keepdims=True) / float(M)
    logits_tag = jnp.dot(hbar, tagWT_ref[...],
                         preferred_element_type=f32) + tagb_ref[...]

    gate = jax.nn.sigmoid(tsc_ref[0, 0])
    out_ref[0] = logits_base + gate * logits_tag


def kernel(x, Wf, bf, Wbase, bbase, proj_W, proj_b, tag_W, tag_b, tag_scale):
    f32 = jnp.float32
    x6 = x.reshape(N, C, T4, 4, V, M)
    xq = jnp.transpose(x, (0, 3, 4, 1, 2)).reshape(N, G, 8, T)
    bf_c = bf.reshape(FEAT_DIM, 1)
    WbaseT = Wbase.T
    bbase_r = bbase.reshape(1, NUM_CLASS)
    projWT = proj_W.T
    projb_r = proj_b.reshape(1, FEAT_DIM)
    tagWT = tag_W.T
    tagb_r = tag_b.reshape(1, NUM_CLASS)
    tsc = tag_scale.reshape(1, 1).astype(f32)

    out3 = pl.pallas_call(
        _fused_kernel,
        grid=(N,),
        in_specs=[
            pl.BlockSpec((1, C, T4, 1, V, M), lambda n: (n, 0, 0, 0, 0, 0)),
            pl.BlockSpec((1, GB, 8, T), lambda n: (n, HIP_L, 0, 0)),
            pl.BlockSpec((1, GB, 8, T), lambda n: (n, HIP_R, 0, 0)),
            pl.BlockSpec((FEAT_DIM, C), lambda n: (0, 0)),
            pl.BlockSpec((FEAT_DIM, 1), lambda n: (0, 0)),
            pl.BlockSpec((FEAT_DIM, NUM_CLASS), lambda n: (0, 0)),
            pl.BlockSpec((1, NUM_CLASS), lambda n: (0, 0)),
            pl.BlockSpec((FEAT_DIM, FEAT_DIM), lambda n: (0, 0)),
            pl.BlockSpec((1, FEAT_DIM), lambda n: (0, 0)),
            pl.BlockSpec((FEAT_DIM, NUM_CLASS), lambda n: (0, 0)),
            pl.BlockSpec((1, NUM_CLASS), lambda n: (0, 0)),
            pl.BlockSpec((1, 1), lambda n: (0, 0)),
        ],
        out_specs=pl.BlockSpec((1, 1, NUM_CLASS), lambda n: (n, 0, 0)),
        out_shape=jax.ShapeDtypeStruct((N, 1, NUM_CLASS), f32),
    )(x6, xq, xq, Wf, bf_c, WbaseT, bbase_r, projWT, projb_r, tagWT, tagb_r,
      tsc)
    return out3.reshape(N, NUM_CLASS)
